# baseline (speedup 1.0000x reference)
import jax
import jax.numpy as jnp
from jax.experimental import pallas as pl

B = 16384
V = 213
D_OUT = 10
BLOCK_B = 2048


def _body(x_ref, wt_ref, b_ref, out_ref):
    out_ref[...] = (
        jnp.dot(x_ref[...], wt_ref[...], preferred_element_type=jnp.float32)
        + b_ref[...]
    )


def kernel(x, W, b):
    wt = W.T
    b2 = b.reshape(1, D_OUT)
    grid = (B // BLOCK_B,)
    return pl.pallas_call(
        _body,
        grid=grid,
        in_specs=[
            pl.BlockSpec((BLOCK_B, V), lambda i: (i, 0)),
            pl.BlockSpec((V, D_OUT), lambda i: (0, 0)),
            pl.BlockSpec((1, D_OUT), lambda i: (0, 0)),
        ],
        out_specs=pl.BlockSpec((BLOCK_B, D_OUT), lambda i: (i, 0)),
        out_shape=jax.ShapeDtypeStruct((B, D_OUT), jnp.float32),
    )(x, wt, b2)


# speedup vs baseline: 1.0257x; 1.0257x over previous
import jax
import jax.numpy as jnp
from jax.experimental import pallas as pl

B = 16384
V = 213
D_OUT = 10
BLOCK_B = 16384


def _body(x_ref, wt_ref, b_ref, out_ref):
    out_ref[...] = (
        jnp.dot(x_ref[...], wt_ref[...], preferred_element_type=jnp.float32)
        + b_ref[...]
    )


def kernel(x, W, b):
    wt = W.T
    b2 = b.reshape(1, D_OUT)
    grid = (B // BLOCK_B,)
    return pl.pallas_call(
        _body,
        grid=grid,
        in_specs=[
            pl.BlockSpec((BLOCK_B, V), lambda i: (i, 0)),
            pl.BlockSpec((V, D_OUT), lambda i: (0, 0)),
            pl.BlockSpec((1, D_OUT), lambda i: (0, 0)),
        ],
        out_specs=pl.BlockSpec((BLOCK_B, D_OUT), lambda i: (i, 0)),
        out_shape=jax.ShapeDtypeStruct((B, D_OUT), jnp.float32),
    )(x, wt, b2)


# speedup vs baseline: 1.0582x; 1.0317x over previous
import jax
import jax.numpy as jnp
from jax.experimental import pallas as pl
from jax.experimental.pallas import tpu as pltpu

B = 16384
V = 213
D_OUT = 10
NCHUNK = 8
CH = B // NCHUNK


def _body(x_hbm, wt_ref, b_ref, out_hbm, xbuf, obuf, insem, outsem):
    in_cps = []
    for k in range(NCHUNK):
        cp = pltpu.make_async_copy(
            x_hbm.at[pl.ds(k * CH, CH), :], xbuf.at[k], insem.at[k]
        )
        cp.start()
        in_cps.append(cp)
    out_cps = []
    for k in range(NCHUNK):
        in_cps[k].wait()
        obuf[k] = (
            jnp.dot(xbuf[k], wt_ref[...], preferred_element_type=jnp.float32)
            + b_ref[...]
        )
        cp = pltpu.make_async_copy(
            obuf.at[k], out_hbm.at[pl.ds(k * CH, CH), :], outsem.at[k]
        )
        cp.start()
        out_cps.append(cp)
    for k in range(NCHUNK):
        out_cps[k].wait()


def kernel(x, W, b):
    wt = W.T
    b2 = b.reshape(1, D_OUT)
    return pl.pallas_call(
        _body,
        in_specs=[
            pl.BlockSpec(memory_space=pl.ANY),
            pl.BlockSpec((V, D_OUT), lambda: (0, 0)),
            pl.BlockSpec((1, D_OUT), lambda: (0, 0)),
        ],
        out_specs=pl.BlockSpec(memory_space=pl.ANY),
        out_shape=jax.ShapeDtypeStruct((B, D_OUT), jnp.float32),
        scratch_shapes=[
            pltpu.VMEM((NCHUNK, CH, V), jnp.float32),
            pltpu.VMEM((NCHUNK, CH, D_OUT), jnp.float32),
            pltpu.SemaphoreType.DMA((NCHUNK,)),
            pltpu.SemaphoreType.DMA((NCHUNK,)),
        ],
    )(x, wt, b2)


# speedup vs baseline: 3.3880x; 3.2018x over previous
import jax
import jax.numpy as jnp
from jax.experimental import pallas as pl

B = 16384
V = 213
D_OUT = 10
BLOCK_N = 2048


def _body(xt_ref, w_ref, b_ref, out_ref):
    out_ref[...] = (
        jnp.dot(w_ref[...], xt_ref[...], preferred_element_type=jnp.float32)
        + b_ref[...]
    )


def kernel(x, W, b):
    xt = x.T
    b2 = b.reshape(D_OUT, 1)
    outT = pl.pallas_call(
        _body,
        grid=(B // BLOCK_N,),
        in_specs=[
            pl.BlockSpec((V, BLOCK_N), lambda i: (0, i)),
            pl.BlockSpec((D_OUT, V), lambda i: (0, 0)),
            pl.BlockSpec((D_OUT, 1), lambda i: (0, 0)),
        ],
        out_specs=pl.BlockSpec((D_OUT, BLOCK_N), lambda i: (0, i)),
        out_shape=jax.ShapeDtypeStruct((D_OUT, B), jnp.float32),
    )(xt, W, b2)
    return outT.T
